# baseline (speedup 1.0000x reference)
import functools

import jax
import jax.numpy as jnp
from jax import lax
from jax.experimental import pallas as pl
from jax.experimental.pallas import tpu as pltpu
from jax.experimental.pallas import tpu_sc as plsc

D = 64
SCALE = 8.0
CHUNK = 128
LANES = 16
NBUF = 8
K = NBUF - 2


@functools.cache
def _build(n_ids: int):
    info = plsc.get_sparse_core_info()
    nc, ns = info.num_cores, info.num_subcores
    nw = nc * ns
    assert n_ids % (nw * CHUNK * NBUF) == 0
    per_w = n_ids // nw
    chunks = per_w // CHUNK

    mesh = plsc.VectorSubcoreMesh(core_axis_name="c", subcore_axis_name="s")

    @functools.partial(
        pl.kernel,
        mesh=mesh,
        compiler_params=pltpu.CompilerParams(use_tc_tiling_on_sc=False),
        out_type=jax.ShapeDtypeStruct((n_ids, D), jnp.float32),
        scratch_types=(
            [pltpu.VMEM((chunks, CHUNK), jnp.int32)]
            + [pltpu.VMEM((CHUNK, D), jnp.float32)] * NBUF
            + [pltpu.SemaphoreType.DMA] * (2 * NBUF)
        ),
    )
    def k(ids_hbm, table_hbm, out_hbm, idx_all, *rest):
        bufs = rest[:NBUF]
        gsem = rest[NBUF:2 * NBUF]
        ssem = rest[2 * NBUF:]

        wid = lax.axis_index("s") * nc + lax.axis_index("c")
        base = wid * per_w

        pltpu.sync_copy(ids_hbm.at[pl.ds(wid * chunks, chunks)], idx_all)

        for c in range(K):
            pltpu.async_copy(table_hbm.at[idx_all.at[c]], bufs[c], gsem[c])

        def outer(t, carry):
            for b in range(NBUF):
                g = t * NBUF + b

                sf = (b + K) % NBUF

                @pl.when(jnp.logical_and(g + K < chunks, g >= 2))
                def _wait_store():
                    pltpu.make_async_copy(
                        bufs[sf], out_hbm.at[pl.ds(base, CHUNK)], ssem[sf]
                    ).wait()

                @pl.when(g + K < chunks)
                def _fire_gather():
                    pltpu.async_copy(
                        table_hbm.at[idx_all.at[g + K]], bufs[sf], gsem[sf]
                    )

                pltpu.make_async_copy(
                    table_hbm.at[idx_all.at[g]], bufs[b], gsem[b]
                ).wait()

                def scale_row(r, c2):
                    for c in range(D // LANES):
                        sl = pl.ds(c * LANES, LANES)
                        bufs[b][r, sl] = bufs[b][r, sl] * SCALE
                    return c2

                lax.fori_loop(0, CHUNK, scale_row, 0)
                pltpu.async_copy(
                    bufs[b], out_hbm.at[pl.ds(base + g * CHUNK, CHUNK)], ssem[b]
                )
            return carry

        lax.fori_loop(0, chunks // NBUF, outer, 0)

        for b in range(NBUF):
            pltpu.make_async_copy(
                bufs[b], out_hbm.at[pl.ds(base, CHUNK)], ssem[b]
            ).wait()

    return k


def kernel(input_ids, table):
    b, h = input_ids.shape
    n_ids = b * h
    ids2d = input_ids.reshape(n_ids // CHUNK, CHUNK).astype(jnp.int32)
    out = _build(n_ids)(ids2d, table)
    return out.reshape(b, h, D)
